# baseline (speedup 1.0000x reference)
import functools

import jax
import jax.numpy as jnp
from jax import lax
from jax.experimental import pallas as pl
from jax.experimental.pallas import tpu as pltpu
from jax.experimental.pallas import tpu_sc as plsc

NUM_NODES = 1000000
EMBED_DIM = 64
BATCH = 16384

_info = plsc.get_sparse_core_info()
_NC, _NS = _info.num_cores, _info.num_subcores
_NW = _NC * _NS
_BLK = 128
_NCOL = (NUM_NODES + _BLK - 1) // _BLK
_CPW = 256
_CAND = BATCH + 16

_mesh = plsc.VectorSubcoreMesh(core_axis_name="c", subcore_axis_name="s")

@functools.partial(
    pl.kernel,
    mesh=_mesh,
    out_type=jax.ShapeDtypeStruct((_CAND, _BLK), jnp.float32),
    scratch_types=[
        pltpu.VMEM((BATCH,), jnp.int32),
        pltpu.VMEM((BATCH,), jnp.int32),
        pltpu.VMEM((BATCH,), jnp.int32),
        pltpu.VMEM((BATCH,), jnp.int32),
        pltpu.VMEM((BATCH,), jnp.int32),
        pltpu.VMEM((_CPW,), jnp.int32),
        pltpu.VMEM((_CPW,), jnp.int32),
        *[pltpu.VMEM((EMBED_DIM, _BLK), jnp.float32) for _ in range(4)],
        pltpu.VMEM((16, _BLK), jnp.float32),
        pltpu.VMEM((16,), jnp.int32),
        *[pltpu.SemaphoreType.DMA for _ in range(5)],
    ],
    compiler_params=pltpu.CompilerParams(needs_layout_passes=False),
)
def _dedup_kernel(idx_hbm, tableT_hbm, cand_hbm, idxg, lr_v, lj_v, qr_v, qj_v,
                  bmap, clist, t0, t1, t2, t3, mrows, midx,
                  s0, s1, s2, s3, semw):
    w = lax.axis_index("s") * _NC + lax.axis_index("c")
    lane = lax.iota(jnp.int32, 16)
    ones = lane * 0 + 1
    trash = lane * 0 + BATCH
    c16 = [lane + 16 * k for k in range(EMBED_DIM // 16)]
    tbufs = (t0, t1, t2, t3)
    sems = (s0, s1, s2, s3)

    pltpu.sync_copy(idx_hbm, idxg)
    for q in range(_CPW // 16):
        bmap[pl.ds(16 * q, 16)] = lane * 0
    midx[pl.ds(0, 16)] = trash

    def scan_body(i, nm):
        v = idxg[pl.ds(16 * i, 16)]
        col = lax.shift_right_logical(v, 7)
        mine = lax.shift_right_logical(col, 8) == w
        plsc.store_scatter(bmap, [col & (_CPW - 1)], ones, mask=mine)
        pos = plsc.cumsum(jnp.where(mine, 1, 0))
        plsc.store_scatter(lr_v, [nm + pos - 1], v, mask=mine)
        plsc.store_scatter(lj_v, [nm + pos - 1], lane + 16 * i, mask=mine)
        return nm + plsc.all_reduce_population_count(mine)[0]

    nm = lax.fori_loop(0, BATCH // 16, scan_body, jnp.int32(0))

    def cmp_body(q, nc):
        f = bmap[pl.ds(16 * q, 16)] > 0
        pos = plsc.cumsum(jnp.where(f, 1, 0))
        plsc.store_scatter(clist, [nc + pos - 1], lane + 16 * q, mask=f)
        return nc + plsc.all_reduce_population_count(f)[0]

    nc = lax.fori_loop(0, _CPW // 16, cmp_body, jnp.int32(0))

    def at_scalar(ref, p):
        vec = ref[pl.ds(lax.shift_right_logical(p, 4) * 16, 16)]
        return jnp.sum(jnp.where(lane == (p & 15), vec, 0))

    def fetch(colg, b):
        rbase = pl.multiple_of(colg * _BLK, _BLK)
        pltpu.async_copy(tableT_hbm.at[:, pl.ds(rbase, _BLK)],
                         tbufs[b], sems[b])

    def wait_fetch(b):
        pltpu.make_async_copy(tableT_hbm.at[:, pl.ds(0, _BLK)],
                              tbufs[b], sems[b]).wait()

    def flush():
        pltpu.async_copy(mrows, cand_hbm.at[midx], semw)
        pltpu.make_async_copy(mrows, cand_hbm.at[midx], semw).wait()
        midx[pl.ds(0, 16)] = trash

    for d in range(3):
        @pl.when(d < nc)
        def _():
            fetch(at_scalar(clist, jnp.int32(d)) + _CPW * w, d)

    def col_body(k, mtot):
        for b in range(4):
            i = 4 * k + b

            def process(mtot):
                colg = at_scalar(clist, i) + _CPW * w
                wait_fetch(b)

                @pl.when(i + 3 < nc)
                def _():
                    fetch(at_scalar(clist, i + 3) + _CPW * w, (b + 3) & 3)

                def match_body(m, qn):
                    rv = lr_v[pl.ds(16 * m, 16)]
                    jv = lj_v[pl.ds(16 * m, 16)]
                    inb = (lane + 16 * m) < nm
                    mm = (lax.shift_right_logical(rv, 7) == colg) & inb
                    pos = plsc.cumsum(jnp.where(mm, 1, 0))
                    plsc.store_scatter(qr_v, [qn + pos - 1], rv, mask=mm)
                    plsc.store_scatter(qj_v, [qn + pos - 1], jv, mask=mm)
                    return qn + plsc.all_reduce_population_count(mm)[0]

                qn = lax.fori_loop(0, (nm + 15) // 16, match_body,
                                   jnp.int32(0))

                flat = tbufs[b].reshape(1, EMBED_DIM * _BLK).at[0]

                def drain_body(e, mtot):
                    r = at_scalar(qr_v, e)
                    j = at_scalar(qj_v, e)
                    slot = mtot & 15
                    rr = (lane & 0) + (r & (_BLK - 1))
                    for kk in range(EMBED_DIM // 16):
                        vals = plsc.load_gather(flat, [c16[kk] * _BLK + rr])
                        mrows[slot, pl.ds(16 * kk, 16)] = vals
                    plsc.store_scatter(midx, [(lane & 0) + slot],
                                       (lane & 0) + j, mask=lane == 0)

                    @pl.when(slot == 15)
                    def _():
                        flush()
                    return mtot + 1

                return lax.fori_loop(0, qn, drain_body, mtot)

            mtot = lax.cond(i < nc, process, lambda m: m, mtot)
        return mtot

    mtot = lax.fori_loop(0, (_CPW + 3) // 4, col_body, jnp.int32(0))

    @pl.when((mtot & 15) != 0)
    def _():
        flush()


def kernel(nodes, ent_features):
    cand = _dedup_kernel(nodes.astype(jnp.int32), ent_features.T)
    return cand[:BATCH, :EMBED_DIM]
